# baseline (speedup 1.0000x reference)
import jax
import jax.numpy as jnp
from jax.experimental import pallas as pl

KTOP = 64


def _enc_body(x_ref, w_ref, b_ref, out_ref):
    w_bf = w_ref[...].astype(jnp.bfloat16)
    acc = jax.lax.dot_general(
        x_ref[...], w_bf, (((1,), (1,)), ((), ())),
        preferred_element_type=jnp.float32)
    out_ref[...] = acc + b_ref[...]


def _encoder(xc_bf, W_enc, b_enc, bh):
    m, d = xc_bf.shape
    h = W_enc.shape[0]
    return pl.pallas_call(
        _enc_body,
        grid=(h // bh,),
        in_specs=[
            pl.BlockSpec((m, d), lambda j: (0, 0)),
            pl.BlockSpec((bh, d), lambda j: (j, 0)),
            pl.BlockSpec((1, bh), lambda j: (0, j)),
        ],
        out_specs=pl.BlockSpec((m, bh), lambda j: (0, j)),
        out_shape=jax.ShapeDtypeStruct((m, h), jnp.float32),
    )(xc_bf, W_enc, b_enc.reshape(1, h))


def _topk_body(enc_ref, act_ref, act_bf_ref):
    enc = enc_ref[...]
    bm, h = enc.shape
    v = jax.lax.bitcast_convert_type(enc, jnp.int32)
    keys = v ^ ((v >> 31) & jnp.int32(0x7FFFFFFF))

    ngrp = h // 128
    cm = keys[:, :128]
    for c in range(1, ngrp):
        cm = jnp.maximum(cm, keys[:, c * 128:(c + 1) * 128])

    cnt0 = jnp.sum(cm >= 0, axis=1, keepdims=True)
    cur = jnp.where(cnt0 >= KTOP, jnp.int32(0), jnp.int32(-2147483648))

    def mini_body(i, cur):
        bit = jnp.int32(1) << (jnp.int32(30) - i)
        cand = cur | bit
        cnt = jnp.sum(cm >= cand, axis=1, keepdims=True)
        return jnp.where(cnt >= KTOP, cand, cur)

    lo_g = jax.lax.fori_loop(0, 31, mini_body, cur)
    umax = jnp.max(cm, axis=1, keepdims=True)

    cnt_pos = jnp.sum(keys >= 0, axis=1, keepdims=True)
    pos = cnt_pos >= KTOP
    lo0 = jnp.where(pos, jnp.maximum(lo_g, 0), lo_g)
    hi0 = jnp.where(pos, umax, jnp.minimum(umax, -1))
    cl0 = jnp.sum(keys >= lo0, axis=1, keepdims=True)
    ch0 = jnp.where(pos, jnp.zeros_like(cnt_pos), cnt_pos)

    def w_cond(state):
        lo, hi, cl, ch, it = state
        return jnp.any(lo < hi)

    def w_body(state):
        lo, hi, cl, ch, it = state
        act = lo < hi
        span = (hi - lo + 1).astype(jnp.float32)
        frac = (cl - KTOP).astype(jnp.float32) / (cl - ch).astype(jnp.float32)
        interp = lo + (span * frac).astype(jnp.int32)
        bisect = lo + ((hi - lo + 1) >> 1)
        mid = jnp.where(it % 2 == 0, interp, bisect)
        mid = jnp.clip(mid, lo + 1, hi)
        cnt = jnp.sum(keys >= mid, axis=1, keepdims=True)
        take = cnt >= KTOP
        exact = cnt == KTOP
        new_lo = jnp.where(take, mid, lo)
        new_hi = jnp.where(exact, mid, jnp.where(take, hi, mid - 1))
        new_cl = jnp.where(take, cnt, cl)
        new_ch = jnp.where(take, ch, cnt)
        return (jnp.where(act, new_lo, lo), jnp.where(act, new_hi, hi),
                jnp.where(act, new_cl, cl), jnp.where(act, new_ch, ch),
                it + 1)

    thr, _, _, _, _ = jax.lax.while_loop(
        w_cond, w_body, (lo0, hi0, cl0, ch0, jnp.int32(0)))
    activated = jnp.where(keys >= thr, enc, jnp.zeros_like(enc))
    act_ref[...] = activated
    act_bf_ref[...] = activated.astype(jnp.bfloat16)


def _topk_mask(encoded, bm):
    m, h = encoded.shape
    return pl.pallas_call(
        _topk_body,
        grid=(m // bm,),
        in_specs=[pl.BlockSpec((bm, h), lambda i: (i, 0))],
        out_specs=[pl.BlockSpec((bm, h), lambda i: (i, 0)),
                   pl.BlockSpec((bm, h), lambda i: (i, 0))],
        out_shape=[jax.ShapeDtypeStruct((m, h), jnp.float32),
                   jax.ShapeDtypeStruct((m, h), jnp.bfloat16)],
    )(encoded)


def _dec_body(act_ref, w_ref, pb_ref, out_ref):
    k = pl.program_id(1)

    @pl.when(k == 0)
    def _():
        out_ref[...] = jnp.broadcast_to(pb_ref[...], out_ref.shape)

    w_bf = w_ref[...].astype(jnp.bfloat16)
    out_ref[...] += jax.lax.dot_general(
        act_ref[...], w_bf, (((1,), (1,)), ((), ())),
        preferred_element_type=jnp.float32)


def _decoder(act_bf, W_dec, pre_bias, bd, bk):
    m, h = act_bf.shape
    d = W_dec.shape[0]
    return pl.pallas_call(
        _dec_body,
        grid=(d // bd, h // bk),
        in_specs=[
            pl.BlockSpec((m, bk), lambda j, k: (0, k)),
            pl.BlockSpec((bd, bk), lambda j, k: (j, k)),
            pl.BlockSpec((1, bd), lambda j, k: (0, j)),
        ],
        out_specs=pl.BlockSpec((m, bd), lambda j, k: (0, j)),
        out_shape=jax.ShapeDtypeStruct((m, d), jnp.float32),
    )(act_bf, W_dec, pre_bias.reshape(1, d))


def kernel(x, pre_bias, W_enc, b_enc, W_dec):
    m, d = x.shape
    h = W_enc.shape[0]
    bh = min(512, h)
    bm = min(128, m)
    bd = min(1024, d)
    bk = min(1024, h)
    xc_bf = (x - pre_bias[None, :]).astype(jnp.bfloat16)
    encoded = _encoder(xc_bf, W_enc, b_enc, bh)
    activated, act_bf = _topk_mask(encoded, bm)
    decoded = _decoder(act_bf, W_dec, pre_bias, bd, bk)
    return (decoded, activated)


# speedup vs baseline: 1.0213x; 1.0213x over previous
import jax
import jax.numpy as jnp
from jax.experimental import pallas as pl

KTOP = 64


def _enc_body(x_ref, w_ref, b_ref, out_ref):
    w_bf = w_ref[...].astype(jnp.bfloat16)
    acc = jax.lax.dot_general(
        x_ref[...], w_bf, (((1,), (1,)), ((), ())),
        preferred_element_type=jnp.float32)
    out_ref[...] = acc + b_ref[...]


def _encoder(xc_bf, W_enc, b_enc, bh):
    m, d = xc_bf.shape
    h = W_enc.shape[0]
    return pl.pallas_call(
        _enc_body,
        grid=(h // bh,),
        in_specs=[
            pl.BlockSpec((m, d), lambda j: (0, 0)),
            pl.BlockSpec((bh, d), lambda j: (j, 0)),
            pl.BlockSpec((1, bh), lambda j: (0, j)),
        ],
        out_specs=pl.BlockSpec((m, bh), lambda j: (0, j)),
        out_shape=jax.ShapeDtypeStruct((m, h), jnp.float32),
    )(xc_bf, W_enc, b_enc.reshape(1, h))


def _topk_body(enc_ref, act_ref, act_bf_ref):
    enc = enc_ref[...]
    bm, h = enc.shape
    v = jax.lax.bitcast_convert_type(enc, jnp.int32)
    keys = v ^ ((v >> 31) & jnp.int32(0x7FFFFFFF))

    ngrp = h // 128
    cm = keys[:, :128]
    for c in range(1, ngrp):
        cm = jnp.maximum(cm, keys[:, c * 128:(c + 1) * 128])

    cnt0 = jnp.sum(cm >= 0, axis=1, keepdims=True)
    cur = jnp.where(cnt0 >= KTOP, jnp.int32(0), jnp.int32(-2147483648))

    def mini_body(i, cur):
        bit = jnp.int32(1) << (jnp.int32(30) - i)
        cand = cur | bit
        cnt = jnp.sum(cm >= cand, axis=1, keepdims=True)
        return jnp.where(cnt >= KTOP, cand, cur)

    lo0 = jax.lax.fori_loop(0, 31, mini_body, cur)
    hi0 = jnp.max(cm, axis=1, keepdims=True)

    def w_cond(state):
        lo, hi = state
        return jnp.any(lo < hi)

    def w_body(state):
        lo, hi = state
        act = lo < hi
        mid = lo + ((hi - lo + 1) >> 1)
        cnt = jnp.sum(keys >= mid, axis=1, keepdims=True)
        take = cnt >= KTOP
        exact = cnt == KTOP
        new_lo = jnp.where(take, mid, lo)
        new_hi = jnp.where(exact, mid, jnp.where(take, hi, mid - 1))
        return (jnp.where(act, new_lo, lo), jnp.where(act, new_hi, hi))

    thr, _ = jax.lax.while_loop(w_cond, w_body, (lo0, hi0))
    activated = jnp.where(keys >= thr, enc, jnp.zeros_like(enc))
    act_ref[...] = activated
    act_bf_ref[...] = activated.astype(jnp.bfloat16)


def _topk_mask(encoded, bm):
    m, h = encoded.shape
    return pl.pallas_call(
        _topk_body,
        grid=(m // bm,),
        in_specs=[pl.BlockSpec((bm, h), lambda i: (i, 0))],
        out_specs=[pl.BlockSpec((bm, h), lambda i: (i, 0)),
                   pl.BlockSpec((bm, h), lambda i: (i, 0))],
        out_shape=[jax.ShapeDtypeStruct((m, h), jnp.float32),
                   jax.ShapeDtypeStruct((m, h), jnp.bfloat16)],
    )(encoded)


def _dec_body(act_ref, w_ref, pb_ref, out_ref):
    k = pl.program_id(1)

    @pl.when(k == 0)
    def _():
        out_ref[...] = jnp.broadcast_to(pb_ref[...], out_ref.shape)

    w_bf = w_ref[...].astype(jnp.bfloat16)
    out_ref[...] += jax.lax.dot_general(
        act_ref[...], w_bf, (((1,), (1,)), ((), ())),
        preferred_element_type=jnp.float32)


def _decoder(act_bf, W_dec, pre_bias, bd, bk):
    m, h = act_bf.shape
    d = W_dec.shape[0]
    return pl.pallas_call(
        _dec_body,
        grid=(d // bd, h // bk),
        in_specs=[
            pl.BlockSpec((m, bk), lambda j, k: (0, k)),
            pl.BlockSpec((bd, bk), lambda j, k: (j, k)),
            pl.BlockSpec((1, bd), lambda j, k: (0, j)),
        ],
        out_specs=pl.BlockSpec((m, bd), lambda j, k: (0, j)),
        out_shape=jax.ShapeDtypeStruct((m, d), jnp.float32),
    )(act_bf, W_dec, pre_bias.reshape(1, d))


def kernel(x, pre_bias, W_enc, b_enc, W_dec):
    m, d = x.shape
    h = W_enc.shape[0]
    bh = min(512, h)
    bm = min(128, m)
    bd = min(1024, d)
    bk = min(1024, h)
    xc_bf = (x - pre_bias[None, :]).astype(jnp.bfloat16)
    encoded = _encoder(xc_bf, W_enc, b_enc, bh)
    activated, act_bf = _topk_mask(encoded, bm)
    decoded = _decoder(act_bf, W_dec, pre_bias, bd, bk)
    return (decoded, activated)
